# Initial kernel scaffold; baseline (speedup 1.0000x reference)
import jax
import jax.numpy as jnp
from jax.experimental import pallas as pl


def kernel(X, edge_index, edge_weight, W, b):
    raise NotImplementedError("write your pallas kernel here")


# speedup vs baseline: 4.0993x; 4.0993x over previous
import functools

import jax
import jax.numpy as jnp
from jax import lax
from jax.experimental import pallas as pl
from jax.experimental.pallas import tpu as pltpu
from jax.experimental.pallas import tpu_sc as plsc

N_NODES = 10000
N_PAD = 10240
D = 128
N_EDGES = 320000
NC = 2
NS = 16
L = 16
NW = NC * NS
E_PER_W = N_EDGES // NW
CHUNK = 80
N_CHUNKS = E_PER_W // CHUNK
ROWS_PER_SUB = N_PAD // NS


def _sc_gather_scatter(X, src, dst, ew):
    mesh = plsc.VectorSubcoreMesh(core_axis_name="c", subcore_axis_name="s")

    @functools.partial(
        pl.kernel,
        mesh=mesh,
        out_type=jax.ShapeDtypeStruct((NC, N_PAD, D), jnp.float32),
        scratch_types=[
            pltpu.VMEM((CHUNK,), jnp.int32),
            pltpu.VMEM((CHUNK,), jnp.int32),
            pltpu.VMEM((CHUNK,), jnp.float32),
            pltpu.VMEM((CHUNK, D), jnp.float32),
            pltpu.VMEM_SHARED((N_PAD, D), jnp.float32),
            pltpu.SemaphoreType.DMA,
        ],
    )
    def k(x_hbm, src_hbm, dst_hbm, w_hbm, out_hbm, src_v, dst_v, w_v, rows_v,
          acc, sem):
        c = lax.axis_index("c")
        s = lax.axis_index("s")
        wid = c * NS + s

        def zero_row(r, carry):
            for j in range(D // L):
                rows_v[r, pl.ds(j * L, L)] = jnp.zeros((L,), jnp.float32)
            return carry

        lax.fori_loop(0, CHUNK, zero_row, 0)
        for z in range(ROWS_PER_SUB // CHUNK):
            pltpu.sync_copy(rows_v,
                            acc.at[pl.ds(s * ROWS_PER_SUB + z * CHUNK, CHUNK)])
        plsc.subcore_barrier()

        base = wid * E_PER_W

        def chunk_body(ci, carry):
            off = base + ci * CHUNK
            pltpu.sync_copy(src_hbm.at[pl.ds(off, CHUNK)], src_v)
            pltpu.sync_copy(dst_hbm.at[pl.ds(off, CHUNK)], dst_v)
            pltpu.sync_copy(w_hbm.at[pl.ds(off, CHUNK)], w_v)
            pltpu.async_copy(x_hbm.at[src_v], rows_v, sem).wait()

            def group_body(g, gcarry):
                wgrp = w_v[pl.ds(g * L, L)]
                for r in range(L):
                    wvec = wgrp.at[jnp.full((L,), r, jnp.int32)].get(
                        mode="promise_in_bounds")
                    e = g * L + r
                    for j in range(D // L):
                        sl = pl.ds(j * L, L)
                        rows_v[e, sl] = rows_v[e, sl] * wvec
                return gcarry

            lax.fori_loop(0, CHUNK // L, group_body, 0)

            pltpu.sync_copy(rows_v, acc.at[dst_v], add=True)
            return carry

        lax.fori_loop(0, N_CHUNKS, chunk_body, 0)
        plsc.subcore_barrier()

        pltpu.sync_copy(acc.at[pl.ds(s * ROWS_PER_SUB, ROWS_PER_SUB)],
                        out_hbm.at[c, pl.ds(s * ROWS_PER_SUB, ROWS_PER_SUB)])

    return k(X, src, dst, ew)


def _tc_linear_relu(partials, W, b):
    R = 512
    grid = (N_PAD // R,)

    def mm(p_ref, w_ref, b_ref, o_ref):
        h = p_ref[0] + p_ref[1]
        o_ref[...] = jnp.maximum(
            jnp.dot(h, w_ref[...], preferred_element_type=jnp.float32)
            + b_ref[...], 0.0)

    return pl.pallas_call(
        mm,
        grid=grid,
        in_specs=[
            pl.BlockSpec((2, R, D), lambda i: (0, i, 0)),
            pl.BlockSpec((D, D), lambda i: (0, 0)),
            pl.BlockSpec((1, D), lambda i: (0, 0)),
        ],
        out_specs=pl.BlockSpec((R, D), lambda i: (i, 0)),
        out_shape=jax.ShapeDtypeStruct((N_PAD, D), jnp.float32),
    )(partials, W, b.reshape(1, D))


def kernel(X, edge_index, edge_weight, W, b):
    src = edge_index[1].astype(jnp.int32)
    dst = edge_index[0].astype(jnp.int32)
    partials = _sc_gather_scatter(X, src, dst, edge_weight)
    out = _tc_linear_relu(partials, W, b)
    return out[:N_NODES]


# speedup vs baseline: 8.4557x; 2.0627x over previous
import functools

import jax
import jax.numpy as jnp
from jax import lax
from jax.experimental import pallas as pl
from jax.experimental.pallas import tpu as pltpu
from jax.experimental.pallas import tpu_sc as plsc

N_NODES = 10000
N_PAD = 10240
D = 128
N_EDGES = 320000
NC = 2
NS = 16
L = 16
NW = NC * NS
E_PER_W = N_EDGES // NW
CHUNK = 80
N_CHUNKS = E_PER_W // CHUNK
PH = 25
N_PHASES = N_CHUNKS // PH
ROWS_PER_SUB = N_PAD // NS


def _sc_gather_scatter(X, src, dst, ew):
    mesh = plsc.VectorSubcoreMesh(core_axis_name="c", subcore_axis_name="s")

    @functools.partial(
        pl.kernel,
        mesh=mesh,
        out_type=jax.ShapeDtypeStruct((NC, N_PAD, D), jnp.float32),
        scratch_types=[
            pltpu.VMEM((PH, CHUNK), jnp.int32),
            pltpu.VMEM((PH, CHUNK), jnp.int32),
            pltpu.VMEM((PH, CHUNK), jnp.float32),
            pltpu.VMEM((CHUNK, D), jnp.float32),
            pltpu.VMEM((CHUNK, D), jnp.float32),
            pltpu.VMEM_SHARED((N_PAD, D), jnp.float32),
            pltpu.SemaphoreType.DMA,
            pltpu.SemaphoreType.DMA,
        ],
    )
    def k(x_hbm, src_hbm, dst_hbm, w_hbm, out_hbm, src_s, dst_s, w_s,
          rows0, rows1, acc, gsem0, gsem1):
        c = lax.axis_index("c")
        s = lax.axis_index("s")
        wid = c * NS + s

        def zero_row(r, carry):
            for j in range(D // L):
                rows0[r, pl.ds(j * L, L)] = jnp.zeros((L,), jnp.float32)
            return carry

        lax.fori_loop(0, CHUNK, zero_row, 0)
        for z in range(ROWS_PER_SUB // CHUNK):
            pltpu.sync_copy(rows0,
                            acc.at[pl.ds(s * ROWS_PER_SUB + z * CHUNK, CHUNK)])
        plsc.subcore_barrier()

        def scale_rows(ci, rows):
            def group_body(g, gcarry):
                wgrp = w_s[ci, pl.ds(g * L, L)]
                for r in range(L):
                    wvec = wgrp.at[jnp.full((L,), r, jnp.int32)].get(
                        mode="promise_in_bounds")
                    e = g * L + r
                    for j in range(D // L):
                        sl = pl.ds(j * L, L)
                        rows[e, sl] = rows[e, sl] * wvec
                return gcarry

            lax.fori_loop(0, CHUNK // L, group_body, 0)

        def start_gather(ci, rows, gsem):
            pltpu.async_copy(x_hbm.at[src_s.at[ci]], rows, gsem)

        def wait_gather(rows, gsem):
            pltpu.make_async_copy(x_hbm.at[pl.ds(0, CHUNK)], rows, gsem).wait()

        def step(ci, rows, gsem, nrows, ngsem, last=False):
            wait_gather(rows, gsem)
            if not last:
                start_gather(ci + 1, nrows, ngsem)
            scale_rows(ci, rows)
            pltpu.sync_copy(rows, acc.at[dst_s.at[ci]], add=True)

        def phase_body(ph, carry):
            pltpu.sync_copy(src_hbm.at[wid, ph], src_s)
            pltpu.sync_copy(dst_hbm.at[wid, ph], dst_s)
            pltpu.sync_copy(w_hbm.at[wid, ph], w_s)
            start_gather(0, rows0, gsem0)

            def pair_body(p, pcarry):
                step(2 * p, rows0, gsem0, rows1, gsem1)
                step(2 * p + 1, rows1, gsem1, rows0, gsem0)
                return pcarry

            lax.fori_loop(0, (PH - 1) // 2, pair_body, 0)
            step(PH - 1, rows0, gsem0, rows1, gsem1, last=True)
            return carry

        lax.fori_loop(0, N_PHASES, phase_body, 0)
        plsc.subcore_barrier()

        pltpu.sync_copy(acc.at[pl.ds(s * ROWS_PER_SUB, ROWS_PER_SUB)],
                        out_hbm.at[c, pl.ds(s * ROWS_PER_SUB, ROWS_PER_SUB)])

    return k(X, src, dst, ew)


def _tc_linear_relu(partials, W, b):
    R = 512
    grid = (N_PAD // R,)

    def mm(p_ref, w_ref, b_ref, o_ref):
        h = p_ref[0] + p_ref[1]
        o_ref[...] = jnp.maximum(
            jnp.dot(h, w_ref[...], preferred_element_type=jnp.float32)
            + b_ref[...], 0.0)

    return pl.pallas_call(
        mm,
        grid=grid,
        in_specs=[
            pl.BlockSpec((2, R, D), lambda i: (0, i, 0)),
            pl.BlockSpec((D, D), lambda i: (0, 0)),
            pl.BlockSpec((1, D), lambda i: (0, 0)),
        ],
        out_specs=pl.BlockSpec((R, D), lambda i: (i, 0)),
        out_shape=jax.ShapeDtypeStruct((N_PAD, D), jnp.float32),
    )(partials, W, b.reshape(1, D))


def kernel(X, edge_index, edge_weight, W, b):
    src = edge_index[1].astype(jnp.int32).reshape(NW, N_PHASES, PH, CHUNK)
    dst = edge_index[0].astype(jnp.int32).reshape(NW, N_PHASES, PH, CHUNK)
    ew = edge_weight.reshape(NW, N_PHASES, PH, CHUNK)
    partials = _sc_gather_scatter(X, src, dst, ew)
    return _tc_linear_relu(partials, W, b)[:N_NODES]


# speedup vs baseline: 9.0394x; 1.0690x over previous
import functools

import jax
import jax.numpy as jnp
from jax import lax
from jax.experimental import pallas as pl
from jax.experimental.pallas import tpu as pltpu
from jax.experimental.pallas import tpu_sc as plsc

N_NODES = 10000
N_PAD = 10240
D = 128
N_EDGES = 320000
NC = 2
NS = 16
L = 16
NW = NC * NS
E_PER_W = N_EDGES // NW
CHUNK = 80
N_CHUNKS = E_PER_W // CHUNK
PH = 25
N_PHASES = N_CHUNKS // PH
ROWS_PER_SUB = N_PAD // NS


def _sc_gather_scatter(X, edge_index, ew):
    mesh = plsc.VectorSubcoreMesh(core_axis_name="c", subcore_axis_name="s")

    @functools.partial(
        pl.kernel,
        mesh=mesh,
        out_type=jax.ShapeDtypeStruct((NC, N_NODES, D), jnp.float32),
        scratch_types=[
            pltpu.VMEM((PH, CHUNK), jnp.int32),
            pltpu.VMEM((PH, CHUNK), jnp.int32),
            pltpu.VMEM((PH, CHUNK), jnp.float32),
            pltpu.VMEM((CHUNK, D), jnp.float32),
            pltpu.VMEM((CHUNK, D), jnp.float32),
            pltpu.VMEM_SHARED((N_PAD, D), jnp.float32),
            pltpu.SemaphoreType.DMA,
            pltpu.SemaphoreType.DMA,
            pltpu.SemaphoreType.DMA,
            pltpu.SemaphoreType.DMA,
        ],
    )
    def k(x_hbm, e_hbm, w_hbm, out_hbm, src_s, dst_s, w_s,
          rows0, rows1, acc, gsem0, gsem1, ssem0, ssem1):
        c = lax.axis_index("c")
        s = lax.axis_index("s")
        wid = c * NS + s

        def zero_row(r, carry):
            for j in range(D // L):
                rows0[r, pl.ds(j * L, L)] = jnp.zeros((L,), jnp.float32)
            return carry

        lax.fori_loop(0, CHUNK, zero_row, 0)
        for z in range(ROWS_PER_SUB // CHUNK):
            pltpu.sync_copy(rows0,
                            acc.at[pl.ds(s * ROWS_PER_SUB + z * CHUNK, CHUNK)])
        plsc.subcore_barrier()

        def scale_rows(ci, rows):
            def group_body(g, gcarry):
                wgrp = w_s[ci, pl.ds(g * L, L)]
                for r in range(L):
                    wvec = wgrp.at[jnp.full((L,), r, jnp.int32)].get(
                        mode="promise_in_bounds")
                    e = g * L + r
                    for j in range(D // L):
                        sl = pl.ds(j * L, L)
                        rows[e, sl] = rows[e, sl] * wvec
                return gcarry

            lax.fori_loop(0, CHUNK // L, group_body, 0)

        def start_gather(ci, rows, gsem):
            pltpu.async_copy(x_hbm.at[src_s.at[ci]], rows, gsem)

        def wait_gather(rows, gsem):
            pltpu.make_async_copy(x_hbm.at[pl.ds(0, CHUNK)], rows, gsem).wait()

        def wait_scatter(rows, ssem):
            pltpu.make_async_copy(rows, acc.at[pl.ds(0, CHUNK)], ssem).wait()

        def step(ci, rows, gsem, ssem, nrows, ngsem, nssem,
                 first_pair=False, last=False):
            wait_gather(rows, gsem)
            if not last:
                if not first_pair:
                    wait_scatter(nrows, nssem)
                start_gather(ci + 1, nrows, ngsem)
            scale_rows(ci, rows)
            pltpu.async_copy(rows, acc.at[dst_s.at[ci]], ssem, add=True)

        def phase_body(ph, carry):
            pltpu.sync_copy(e_hbm.at[1, wid, ph], src_s)
            pltpu.sync_copy(e_hbm.at[0, wid, ph], dst_s)
            pltpu.sync_copy(w_hbm.at[wid, ph], w_s)
            start_gather(0, rows0, gsem0)
            step(0, rows0, gsem0, ssem0, rows1, gsem1, ssem1, first_pair=True)

            def pair_body(p, pcarry):
                step(2 * p + 1, rows1, gsem1, ssem1, rows0, gsem0, ssem0)
                step(2 * p + 2, rows0, gsem0, ssem0, rows1, gsem1, ssem1)
                return pcarry

            lax.fori_loop(0, (PH - 3) // 2, pair_body, 0)
            step(PH - 2, rows1, gsem1, ssem1, rows0, gsem0, ssem0)
            step(PH - 1, rows0, gsem0, ssem0, rows1, gsem1, ssem1, last=True)
            wait_scatter(rows0, ssem0)
            wait_scatter(rows1, ssem1)
            return carry

        lax.fori_loop(0, N_PHASES, phase_body, 0)
        plsc.subcore_barrier()

        @pl.when(s < NS - 1)
        def _dump_full():
            pltpu.sync_copy(acc.at[pl.ds(s * ROWS_PER_SUB, ROWS_PER_SUB)],
                            out_hbm.at[c, pl.ds(s * ROWS_PER_SUB,
                                                ROWS_PER_SUB)])

        @pl.when(s == NS - 1)
        def _dump_last():
            tail = N_NODES - (NS - 1) * ROWS_PER_SUB
            pltpu.sync_copy(acc.at[pl.ds((NS - 1) * ROWS_PER_SUB, tail)],
                            out_hbm.at[c, pl.ds((NS - 1) * ROWS_PER_SUB,
                                                tail)])

    return k(X, edge_index, ew)


def _tc_linear_relu(partials, W, b):
    R = 400
    grid = (N_NODES // R,)

    def mm(p_ref, w_ref, b_ref, o_ref):
        h = p_ref[0] + p_ref[1]
        o_ref[...] = jnp.maximum(
            jnp.dot(h, w_ref[...], preferred_element_type=jnp.float32)
            + b_ref[...], 0.0)

    return pl.pallas_call(
        mm,
        grid=grid,
        in_specs=[
            pl.BlockSpec((2, R, D), lambda i: (0, i, 0)),
            pl.BlockSpec((D, D), lambda i: (0, 0)),
            pl.BlockSpec((1, D), lambda i: (0, 0)),
        ],
        out_specs=pl.BlockSpec((R, D), lambda i: (i, 0)),
        out_shape=jax.ShapeDtypeStruct((N_NODES, D), jnp.float32),
    )(partials, W, b.reshape(1, D))


def kernel(X, edge_index, edge_weight, W, b):
    eidx = edge_index.astype(jnp.int32).reshape(2, NW, N_PHASES, PH, CHUNK)
    ew = edge_weight.reshape(NW, N_PHASES, PH, CHUNK)
    partials = _sc_gather_scatter(X, eidx, ew)
    return _tc_linear_relu(partials, W, b)


# speedup vs baseline: 9.5322x; 1.0545x over previous
import functools

import jax
import jax.numpy as jnp
from jax import lax
from jax.experimental import pallas as pl
from jax.experimental.pallas import tpu as pltpu
from jax.experimental.pallas import tpu_sc as plsc

N_NODES = 10000
N_PAD = 10240
D = 128
N_EDGES = 320000
NC = 2
NS = 16
L = 16
NW = NC * NS
E_PER_W = N_EDGES // NW
CHUNK = 80
N_CHUNKS = E_PER_W // CHUNK
PH = 25
N_PHASES = N_CHUNKS // PH
ROWS_PER_SUB = N_PAD // NS


def _sc_gather_scatter(X, edge_index, ew):
    mesh = plsc.VectorSubcoreMesh(core_axis_name="c", subcore_axis_name="s")

    @functools.partial(
        pl.kernel,
        mesh=mesh,
        out_type=jax.ShapeDtypeStruct((NC, N_NODES, D), jnp.float32),
        scratch_types=[
            pltpu.VMEM((PH, CHUNK), jnp.int32),
            pltpu.VMEM((PH, CHUNK), jnp.int32),
            pltpu.VMEM((PH, CHUNK), jnp.float32),
            pltpu.VMEM((CHUNK, D), jnp.float32),
            pltpu.VMEM((CHUNK, D), jnp.float32),
            pltpu.VMEM_SHARED((N_PAD, D), jnp.float32),
            pltpu.SemaphoreType.DMA,
            pltpu.SemaphoreType.DMA,
            pltpu.SemaphoreType.DMA,
            pltpu.SemaphoreType.DMA,
        ],
    )
    def k(x_hbm, e_hbm, w_hbm, out_hbm, src_s, dst_s, w_s,
          rows0, rows1, acc, gsem0, gsem1, ssem0, ssem1):
        c = lax.axis_index("c")
        s = lax.axis_index("s")
        wid = c * NS + s

        def zero_row(r, carry):
            for j in range(D // L):
                rows0[r, pl.ds(j * L, L)] = jnp.zeros((L,), jnp.float32)
            return carry

        lax.fori_loop(0, CHUNK, zero_row, 0)
        for z in range(ROWS_PER_SUB // CHUNK):
            pltpu.sync_copy(rows0,
                            acc.at[pl.ds(s * ROWS_PER_SUB + z * CHUNK, CHUNK)])
        plsc.subcore_barrier()

        def scale_rows(ci, rows):
            def group_body(g, gcarry):
                wgrp = w_s[ci, pl.ds(g * L, L)]
                for r in range(L):
                    wvec = wgrp.at[jnp.full((L,), r, jnp.int32)].get(
                        mode="promise_in_bounds")
                    e = g * L + r
                    for j in range(D // L):
                        sl = pl.ds(j * L, L)
                        rows[e, sl] = rows[e, sl] * wvec
                return gcarry

            lax.fori_loop(0, CHUNK // L, group_body, 0)

        def start_gather(ci, rows, gsem):
            pltpu.async_copy(x_hbm.at[src_s.at[ci]], rows, gsem)

        def wait_gather(rows, gsem):
            pltpu.make_async_copy(x_hbm.at[pl.ds(0, CHUNK)], rows, gsem).wait()

        def wait_scatter(rows, ssem):
            pltpu.make_async_copy(rows, acc.at[pl.ds(0, CHUNK)], ssem).wait()

        def step(ci, rows, gsem, ssem, nrows, ngsem, nssem,
                 first_pair=False, last=False):
            wait_gather(rows, gsem)
            if not last:
                if not first_pair:
                    wait_scatter(nrows, nssem)
                start_gather(ci + 1, nrows, ngsem)
            scale_rows(ci, rows)
            pltpu.async_copy(rows, acc.at[dst_s.at[ci]], ssem, add=True)

        def phase_body(ph, carry):
            pltpu.sync_copy(e_hbm.at[1, wid, ph], src_s)
            pltpu.sync_copy(e_hbm.at[0, wid, ph], dst_s)
            pltpu.sync_copy(w_hbm.at[wid, ph], w_s)
            start_gather(0, rows0, gsem0)
            step(0, rows0, gsem0, ssem0, rows1, gsem1, ssem1, first_pair=True)

            def pair_body(p, pcarry):
                step(2 * p + 1, rows1, gsem1, ssem1, rows0, gsem0, ssem0)
                step(2 * p + 2, rows0, gsem0, ssem0, rows1, gsem1, ssem1)
                return pcarry

            lax.fori_loop(0, (PH - 3) // 2, pair_body, 0)
            step(PH - 2, rows1, gsem1, ssem1, rows0, gsem0, ssem0)
            step(PH - 1, rows0, gsem0, ssem0, rows1, gsem1, ssem1, last=True)
            wait_scatter(rows0, ssem0)
            wait_scatter(rows1, ssem1)
            return carry

        lax.fori_loop(0, N_PHASES, phase_body, 0)
        plsc.subcore_barrier()

        @pl.when(s < NS - 1)
        def _dump_full():
            pltpu.sync_copy(acc.at[pl.ds(s * ROWS_PER_SUB, ROWS_PER_SUB)],
                            out_hbm.at[c, pl.ds(s * ROWS_PER_SUB,
                                                ROWS_PER_SUB)])

        @pl.when(s == NS - 1)
        def _dump_last():
            tail = N_NODES - (NS - 1) * ROWS_PER_SUB
            pltpu.sync_copy(acc.at[pl.ds((NS - 1) * ROWS_PER_SUB, tail)],
                            out_hbm.at[c, pl.ds((NS - 1) * ROWS_PER_SUB,
                                                tail)])

    return k(X, edge_index, ew)


def _tc_linear_relu(partials, W, b):
    R = 2000
    grid = (N_NODES // R,)

    def mm(p_ref, w_ref, b_ref, o_ref):
        h = p_ref[0] + p_ref[1]
        o_ref[...] = jnp.maximum(
            jnp.dot(h, w_ref[...], preferred_element_type=jnp.float32)
            + b_ref[...], 0.0)

    return pl.pallas_call(
        mm,
        grid=grid,
        in_specs=[
            pl.BlockSpec((2, R, D), lambda i: (0, i, 0)),
            pl.BlockSpec((D, D), lambda i: (0, 0)),
            pl.BlockSpec((1, D), lambda i: (0, 0)),
        ],
        out_specs=pl.BlockSpec((R, D), lambda i: (i, 0)),
        out_shape=jax.ShapeDtypeStruct((N_NODES, D), jnp.float32),
    )(partials, W, b.reshape(1, D))


def kernel(X, edge_index, edge_weight, W, b):
    eidx = edge_index.astype(jnp.int32).reshape(2, NW, N_PHASES, PH, CHUNK)
    ew = edge_weight.reshape(NW, N_PHASES, PH, CHUNK)
    partials = _sc_gather_scatter(X, eidx, ew)
    return _tc_linear_relu(partials, W, b)


# speedup vs baseline: 9.5802x; 1.0050x over previous
import functools

import jax
import jax.numpy as jnp
from jax import lax
from jax.experimental import pallas as pl
from jax.experimental.pallas import tpu as pltpu
from jax.experimental.pallas import tpu_sc as plsc

N_NODES = 10000
N_PAD = 10240
D = 128
N_EDGES = 320000
NC = 2
NS = 16
L = 16
NW = NC * NS
E_PER_W = N_EDGES // NW
CHUNK = 80
N_CHUNKS = E_PER_W // CHUNK
PH = 25
N_PHASES = N_CHUNKS // PH
ROWS_PER_SUB = N_PAD // NS


def _sc_gather_scatter(X, edge_index, ew):
    mesh = plsc.VectorSubcoreMesh(core_axis_name="c", subcore_axis_name="s")

    @functools.partial(
        pl.kernel,
        mesh=mesh,
        out_type=jax.ShapeDtypeStruct((NC, N_NODES, D), jnp.float32),
        scratch_types=[
            pltpu.VMEM((PH, CHUNK), jnp.int32),
            pltpu.VMEM((PH, CHUNK), jnp.int32),
            pltpu.VMEM((PH, CHUNK), jnp.float32),
            pltpu.VMEM((CHUNK, D), jnp.float32),
            pltpu.VMEM((CHUNK, D), jnp.float32),
            pltpu.VMEM_SHARED((N_PAD, D), jnp.float32),
            pltpu.SemaphoreType.DMA,
            pltpu.SemaphoreType.DMA,
            pltpu.SemaphoreType.DMA,
            pltpu.SemaphoreType.DMA,
        ],
    )
    def k(x_hbm, e_hbm, w_hbm, out_hbm, src_s, dst_s, w_s,
          rows0, rows1, acc, gsem0, gsem1, ssem0, ssem1):
        c = lax.axis_index("c")
        s = lax.axis_index("s")
        wid = c * NS + s

        def zero_row(r, carry):
            for j in range(D // L):
                rows0[r, pl.ds(j * L, L)] = jnp.zeros((L,), jnp.float32)
            return carry

        lax.fori_loop(0, CHUNK, zero_row, 0)
        for z in range(ROWS_PER_SUB // CHUNK):
            pltpu.sync_copy(rows0,
                            acc.at[pl.ds(s * ROWS_PER_SUB + z * CHUNK, CHUNK)])
        plsc.subcore_barrier()

        def scale_rows(ci, rows):
            def group_body(g, gcarry):
                wgrp = w_s[ci, pl.ds(g * L, L)]
                for r in range(L):
                    wvec = wgrp.at[jnp.full((L,), r, jnp.int32)].get(
                        mode="promise_in_bounds")
                    e = g * L + r
                    for j in range(D // L):
                        sl = pl.ds(j * L, L)
                        rows[e, sl] = rows[e, sl] * wvec
                return gcarry

            lax.fori_loop(0, CHUNK // L, group_body, 0)

        def start_gather(ci, rows, gsem):
            pltpu.async_copy(x_hbm.at[src_s.at[ci]], rows, gsem)

        def wait_gather(rows, gsem):
            pltpu.make_async_copy(x_hbm.at[pl.ds(0, CHUNK)], rows, gsem).wait()

        def wait_scatter(rows, ssem):
            pltpu.make_async_copy(rows, acc.at[pl.ds(0, CHUNK)], ssem).wait()

        def step(ci, rows, gsem, ssem, nrows, ngsem, nssem,
                 first_pair=False, last=False):
            wait_gather(rows, gsem)
            if not last:
                if not first_pair:
                    wait_scatter(nrows, nssem)
                start_gather(ci + 1, nrows, ngsem)
            scale_rows(ci, rows)
            pltpu.async_copy(rows, acc.at[dst_s.at[ci]], ssem, add=True)

        def phase_body(ph, carry):
            pltpu.sync_copy(e_hbm.at[1, wid, ph], src_s)
            pltpu.sync_copy(e_hbm.at[0, wid, ph], dst_s)
            pltpu.sync_copy(w_hbm.at[wid, ph], w_s)
            start_gather(0, rows0, gsem0)
            step(0, rows0, gsem0, ssem0, rows1, gsem1, ssem1, first_pair=True)

            def pair_body(p, pcarry):
                step(2 * p + 1, rows1, gsem1, ssem1, rows0, gsem0, ssem0)
                step(2 * p + 2, rows0, gsem0, ssem0, rows1, gsem1, ssem1)
                return pcarry

            lax.fori_loop(0, (PH - 3) // 2, pair_body, 0)
            step(PH - 2, rows1, gsem1, ssem1, rows0, gsem0, ssem0)
            step(PH - 1, rows0, gsem0, ssem0, rows1, gsem1, ssem1, last=True)
            wait_scatter(rows0, ssem0)
            wait_scatter(rows1, ssem1)
            return carry

        lax.fori_loop(0, N_PHASES, phase_body, 0)
        plsc.subcore_barrier()

        @pl.when(s < NS - 1)
        def _dump_full():
            pltpu.sync_copy(acc.at[pl.ds(s * ROWS_PER_SUB, ROWS_PER_SUB)],
                            out_hbm.at[c, pl.ds(s * ROWS_PER_SUB,
                                                ROWS_PER_SUB)])

        @pl.when(s == NS - 1)
        def _dump_last():
            tail = N_NODES - (NS - 1) * ROWS_PER_SUB
            pltpu.sync_copy(acc.at[pl.ds((NS - 1) * ROWS_PER_SUB, tail)],
                            out_hbm.at[c, pl.ds((NS - 1) * ROWS_PER_SUB,
                                                tail)])

    return k(X, edge_index, ew)


def _tc_linear_relu(partials, W, b):
    R = 10000
    grid = (N_NODES // R,)

    def mm(p_ref, w_ref, b_ref, o_ref):
        h = p_ref[0] + p_ref[1]
        o_ref[...] = jnp.maximum(
            jnp.dot(h, w_ref[...], preferred_element_type=jnp.float32)
            + b_ref[...], 0.0)

    return pl.pallas_call(
        mm,
        grid=grid,
        in_specs=[
            pl.BlockSpec((2, R, D), lambda i: (0, i, 0)),
            pl.BlockSpec((D, D), lambda i: (0, 0)),
            pl.BlockSpec((1, D), lambda i: (0, 0)),
        ],
        out_specs=pl.BlockSpec((R, D), lambda i: (i, 0)),
        out_shape=jax.ShapeDtypeStruct((N_NODES, D), jnp.float32),
    )(partials, W, b.reshape(1, D))


def kernel(X, edge_index, edge_weight, W, b):
    eidx = edge_index.astype(jnp.int32).reshape(2, NW, N_PHASES, PH, CHUNK)
    ew = edge_weight.reshape(NW, N_PHASES, PH, CHUNK)
    partials = _sc_gather_scatter(X, eidx, ew)
    return _tc_linear_relu(partials, W, b)
